# baseline (speedup 1.0000x reference)
import functools

import jax
import jax.numpy as jnp
from jax import lax
from jax.experimental import pallas as pl
from jax.experimental.pallas import tpu as pltpu
from jax.experimental.pallas import tpu_sc as plsc

EMBED_DIM = 768
BATCH = 4
SEQ = 2048

NUM_CORES = 2
NUM_SUBCORES = 16
NUM_WORKERS = NUM_CORES * NUM_SUBCORES
SLICE = SEQ // NUM_WORKERS
CHUNK = 16
QSTEPS = SLICE // CHUNK
VECS = EMBED_DIM // 16
POS0 = 2 * BATCH * CHUNK


def _emb_body(x_hbm, pos_hbm, table_hbm, out_hbm, idx_all, buf,
              isem, psem0, psem1, gsem0, gsem1, ssem0, ssem1):
    psem = (psem0, psem1)
    gsem = (gsem0, gsem1)
    ssem = (ssem0, ssem1)

    wid = lax.axis_index("s") * NUM_CORES + lax.axis_index("c")
    seq0 = wid * SLICE

    idx_cps = [
        pltpu.async_copy(x_hbm.at[b, pl.ds(seq0, SLICE)], idx_all.at[b], isem)
        for b in range(BATCH)
    ]
    for cp in idx_cps:
        cp.wait()

    def issue_q(q):
        p = q % 2
        pos_h = pltpu.async_copy(
            pos_hbm.at[pl.ds(seq0 + q * CHUNK, CHUNK), :],
            buf.at[pl.ds(POS0 + p * CHUNK, CHUNK), :], psem[p])
        g_h = [
            pltpu.async_copy(
                table_hbm.at[idx_all.at[b, pl.ds(q * CHUNK, CHUNK)]],
                buf.at[pl.ds((p * BATCH + b) * CHUNK, CHUNK), :], gsem[p])
            for b in range(BATCH)
        ]
        return pos_h, g_h

    def issue_stores(q):
        p = q % 2
        return [
            pltpu.async_copy(
                buf.at[pl.ds((p * BATCH + b) * CHUNK, CHUNK), :],
                out_hbm.at[pl.ds(b * SEQ + seq0 + q * CHUNK, CHUNK), :],
                ssem[p])
            for b in range(BATCH)
        ]

    hs = {0: issue_q(0), 1: issue_q(1)}
    st = {}
    for q in range(QSTEPS):
        p = q % 2
        pos_h, g_h = hs[q]
        pos_h.wait()
        for h in g_h:
            h.wait()

        posrow = POS0 + p * CHUNK
        rowbase = p * BATCH * CHUNK

        def add_row(r, carry):
            for c in range(VECS):
                sl = pl.ds(c * 16, 16)
                v = buf[posrow + r, sl]
                plsc.addupdate(buf.at[rowbase + r, sl], v)
                plsc.addupdate(buf.at[rowbase + CHUNK + r, sl], v)
                plsc.addupdate(buf.at[rowbase + 2 * CHUNK + r, sl], v)
                plsc.addupdate(buf.at[rowbase + 3 * CHUNK + r, sl], v)
            return carry

        lax.fori_loop(0, CHUNK, add_row, 0)
        st[q] = issue_stores(q)
        if q >= 1 and q + 1 < QSTEPS:
            for h in st[q - 1]:
                h.wait()
            hs[q + 1] = issue_q(q + 1)

    for q in (QSTEPS - 2, QSTEPS - 1):
        for h in st[q]:
            h.wait()


@jax.jit
def _emb(x2d, pos2d, table):
    mesh = plsc.VectorSubcoreMesh(core_axis_name="c", subcore_axis_name="s")
    run = functools.partial(
        pl.kernel,
        out_type=jax.ShapeDtypeStruct((BATCH * SEQ, EMBED_DIM), jnp.float32),
        mesh=mesh,
        scratch_types=[
            pltpu.VMEM((BATCH, SLICE), jnp.int32),
            pltpu.VMEM((POS0 + 2 * CHUNK, EMBED_DIM), jnp.float32),
        ] + [pltpu.SemaphoreType.DMA] * 7,
    )(_emb_body)
    return run(x2d, pos2d, table)


def kernel(x, token_table, position_embedding):
    x2d = x.astype(jnp.int32)
    pos2d = position_embedding[0, : x.shape[1], :]
    out = _emb(x2d, pos2d, token_table)
    return out.reshape(x.shape[0], x.shape[1], EMBED_DIM)


# speedup vs baseline: 1.0508x; 1.0508x over previous
import functools

import jax
import jax.numpy as jnp
from jax import lax
from jax.experimental import pallas as pl
from jax.experimental.pallas import tpu as pltpu
from jax.experimental.pallas import tpu_sc as plsc

EMBED_DIM = 768
BATCH = 4
SEQ = 2048

NUM_CORES = 2
NUM_SUBCORES = 16
NUM_WORKERS = NUM_CORES * NUM_SUBCORES
SLICE = SEQ // NUM_WORKERS
CHUNK = 16
QSTEPS = SLICE // CHUNK
VECS = EMBED_DIM // 16


def _emb_body(x_hbm, pos_hbm, table_hbm, out_hbm, *scr):
    idx_all = scr[0]
    pb = scr[1:3]
    rb = scr[3:11]
    isem = scr[11]
    psem = scr[12:14]
    gsem = scr[14:16]
    ssem = scr[16:18]

    wid = lax.axis_index("s") * NUM_CORES + lax.axis_index("c")
    seq0 = wid * SLICE

    idx_cps = [
        pltpu.async_copy(x_hbm.at[b, pl.ds(seq0, SLICE)], idx_all.at[b], isem)
        for b in range(BATCH)
    ]
    for cp in idx_cps:
        cp.wait()

    def issue_q(q):
        p = q % 2
        pos_h = pltpu.async_copy(
            pos_hbm.at[pl.ds(seq0 + q * CHUNK, CHUNK), :], pb[p], psem[p])
        g_h = [
            pltpu.async_copy(
                table_hbm.at[idx_all.at[b, pl.ds(q * CHUNK, CHUNK)]],
                rb[p * 4 + b], gsem[p])
            for b in range(BATCH)
        ]
        return pos_h, g_h

    def issue_stores(q):
        p = q % 2
        return [
            pltpu.async_copy(
                rb[p * 4 + b],
                out_hbm.at[pl.ds(b * SEQ + seq0 + q * CHUNK, CHUNK), :],
                ssem[p])
            for b in range(BATCH)
        ]

    hs = {0: issue_q(0), 1: issue_q(1)}
    st = {}
    for q in range(QSTEPS):
        p = q % 2
        pos_h, g_h = hs[q]
        pos_h.wait()
        for h in g_h:
            h.wait()

        pbuf = pb[p]
        r0, r1, r2, r3 = rb[p * 4:p * 4 + 4]

        def add_row(r, carry):
            for c in range(VECS):
                sl = pl.ds(c * 16, 16)
                v = pbuf[r, sl]
                plsc.addupdate(r0.at[r, sl], v)
                plsc.addupdate(r1.at[r, sl], v)
                plsc.addupdate(r2.at[r, sl], v)
                plsc.addupdate(r3.at[r, sl], v)
            return carry

        lax.fori_loop(0, CHUNK, add_row, 0)
        st[q] = issue_stores(q)
        if q >= 1 and q + 1 < QSTEPS:
            for h in st[q - 1]:
                h.wait()
            hs[q + 1] = issue_q(q + 1)

    for q in (QSTEPS - 2, QSTEPS - 1):
        for h in st[q]:
            h.wait()


@jax.jit
def _emb(x2d, pos2d, table):
    mesh = plsc.VectorSubcoreMesh(core_axis_name="c", subcore_axis_name="s")
    run = functools.partial(
        pl.kernel,
        out_type=jax.ShapeDtypeStruct((BATCH * SEQ, EMBED_DIM), jnp.float32),
        mesh=mesh,
        scratch_types=[
            pltpu.VMEM((BATCH, SLICE), jnp.int32),
        ] + [pltpu.VMEM((CHUNK, EMBED_DIM), jnp.float32)] * 2
        + [pltpu.VMEM((CHUNK, EMBED_DIM), jnp.float32)] * 8
        + [pltpu.SemaphoreType.DMA] * 7,
    )(_emb_body)
    return run(x2d, pos2d, table)


def kernel(x, token_table, position_embedding):
    x2d = x.astype(jnp.int32)
    pos2d = position_embedding[0, : x.shape[1], :]
    out = _emb(x2d, pos2d, token_table)
    return out.reshape(x.shape[0], x.shape[1], EMBED_DIM)


# speedup vs baseline: 1.0623x; 1.0110x over previous
import functools

import jax
import jax.numpy as jnp
from jax import lax
from jax.experimental import pallas as pl
from jax.experimental.pallas import tpu as pltpu
from jax.experimental.pallas import tpu_sc as plsc

EMBED_DIM = 768
BATCH = 4
SEQ = 2048

NUM_CORES = 2
NUM_SUBCORES = 16
NUM_WORKERS = NUM_CORES * NUM_SUBCORES
SLICE = SEQ // NUM_WORKERS
CHUNK = 16
QSTEPS = SLICE // CHUNK
VECS = EMBED_DIM // 16


def _emb_body(x_hbm, pos_hbm, table_hbm, out_hbm, *scr):
    idx_all = scr[0]
    pb = scr[1:3]
    rb = scr[3:11]
    isem = scr[11]
    psem = scr[12:14]
    gsem = scr[14:16]
    ssem = scr[16:18]

    wid = lax.axis_index("s") * NUM_CORES + lax.axis_index("c")
    seq0 = wid * SLICE

    def issue_pos(q):
        p = q % 2
        return pltpu.async_copy(
            pos_hbm.at[pl.ds(seq0 + q * CHUNK, CHUNK), :], pb[p], psem[p])

    def issue_gathers(q):
        p = q % 2
        return [
            pltpu.async_copy(
                table_hbm.at[idx_all.at[b, pl.ds(q * CHUNK, CHUNK)]],
                rb[p * 4 + b], gsem[p])
            for b in range(BATCH)
        ]

    pos_h0 = issue_pos(0)
    pos_h1 = issue_pos(1)
    idx_cps = [
        pltpu.async_copy(x_hbm.at[b, pl.ds(seq0, SLICE)], idx_all.at[b], isem)
        for b in range(BATCH)
    ]
    for cp in idx_cps:
        cp.wait()

    def issue_q(q):
        return issue_pos(q), issue_gathers(q)

    def issue_stores(q):
        p = q % 2
        return [
            pltpu.async_copy(
                rb[p * 4 + b],
                out_hbm.at[pl.ds(b * SEQ + seq0 + q * CHUNK, CHUNK), :],
                ssem[p])
            for b in range(BATCH)
        ]

    hs = {0: (pos_h0, issue_gathers(0)), 1: (pos_h1, issue_gathers(1))}
    st = {}
    for q in range(QSTEPS):
        p = q % 2
        pos_h, g_h = hs[q]
        pos_h.wait()
        for h in g_h:
            h.wait()

        pbuf = pb[p]
        r0, r1, r2, r3 = rb[p * 4:p * 4 + 4]

        def add_row(r, carry):
            for c in range(VECS):
                sl = pl.ds(c * 16, 16)
                v = pbuf[r, sl]
                plsc.addupdate(r0.at[r, sl], v)
                plsc.addupdate(r1.at[r, sl], v)
                plsc.addupdate(r2.at[r, sl], v)
                plsc.addupdate(r3.at[r, sl], v)
            return carry

        lax.fori_loop(0, CHUNK, add_row, 0)
        st[q] = issue_stores(q)
        if q >= 1 and q + 1 < QSTEPS:
            for h in st[q - 1]:
                h.wait()
            hs[q + 1] = issue_q(q + 1)

    for q in (QSTEPS - 2, QSTEPS - 1):
        for h in st[q]:
            h.wait()


@jax.jit
def _emb(x2d, pos2d, table):
    mesh = plsc.VectorSubcoreMesh(core_axis_name="c", subcore_axis_name="s")
    run = functools.partial(
        pl.kernel,
        out_type=jax.ShapeDtypeStruct((BATCH * SEQ, EMBED_DIM), jnp.float32),
        mesh=mesh,
        scratch_types=[
            pltpu.VMEM((BATCH, SLICE), jnp.int32),
        ] + [pltpu.VMEM((CHUNK, EMBED_DIM), jnp.float32)] * 2
        + [pltpu.VMEM((CHUNK, EMBED_DIM), jnp.float32)] * 8
        + [pltpu.SemaphoreType.DMA] * 7,
    )(_emb_body)
    return run(x2d, pos2d, table)


def kernel(x, token_table, position_embedding):
    x2d = x.astype(jnp.int32)
    pos2d = position_embedding[0, : x.shape[1], :]
    out = _emb(x2d, pos2d, token_table)
    return out.reshape(x.shape[0], x.shape[1], EMBED_DIM)
